# Initial kernel scaffold; baseline (speedup 1.0000x reference)
import jax
import jax.numpy as jnp
from jax.experimental import pallas as pl


def kernel(user_id, hist_item, hist_len, target_item, user_table, item_table, W1, b1, W2, b2):
    raise NotImplementedError("write your pallas kernel here")


# speedup vs baseline: 5.4206x; 5.4206x over previous
import functools

import jax
import jax.numpy as jnp
from jax import lax
from jax.experimental import pallas as pl
from jax.experimental.pallas import tpu as pltpu
from jax.experimental.pallas import tpu_sc as plsc

B = 16384
L = 200
D = 16
NC = 2
NS = 16
NW = NC * NS
BPW = B // NW
G = 16
NG = BPW // G
CHUNK = 128
NCH = (G * L) // CHUNK

_sc_mesh = plsc.VectorSubcoreMesh(core_axis_name="c", subcore_axis_name="s")


@functools.partial(
    pl.kernel,
    mesh=_sc_mesh,
    compiler_params=pltpu.CompilerParams(use_tc_tiling_on_sc=False),
    out_type=(
        jax.ShapeDtypeStruct((B, D), jnp.float32),
        jax.ShapeDtypeStruct((B, D), jnp.float32),
        jax.ShapeDtypeStruct((B, D), jnp.float32),
    ),
    scratch_types=[
        pltpu.VMEM((G * L,), jnp.int32),
        pltpu.VMEM((G * L, D), jnp.float32),
        pltpu.VMEM((G, D), jnp.float32),
        pltpu.VMEM((BPW,), jnp.int32),
        pltpu.VMEM((BPW, D), jnp.float32),
        pltpu.SemaphoreType.DMA,
    ],
)
def _sc_gather_pool(hist_flat, user_id, tgt_id, item_tab, user_tab,
                    hist_sum_out, user_emb_out, tgt_emb_out,
                    idx_v, rows_v, acc_v, uidx_v, urows_v, sem):
    wid = lax.axis_index("s") * NC + lax.axis_index("c")
    base = wid * BPW

    pltpu.sync_copy(user_id.at[pl.ds(base, BPW)], uidx_v)
    cps = [
        pltpu.async_copy(
            user_tab.at[idx_v_c], urows_v.at[pl.ds(c * CHUNK, CHUNK)], sem)
        for c in range(BPW // CHUNK)
        for idx_v_c in (uidx_v.at[pl.ds(c * CHUNK, CHUNK)],)
    ]
    for cp in cps:
        cp.wait()
    pltpu.sync_copy(urows_v, user_emb_out.at[pl.ds(base, BPW)])

    pltpu.sync_copy(tgt_id.at[pl.ds(base, BPW)], uidx_v)
    cps = [
        pltpu.async_copy(
            item_tab.at[idx_v_c], urows_v.at[pl.ds(c * CHUNK, CHUNK)], sem)
        for c in range(BPW // CHUNK)
        for idx_v_c in (uidx_v.at[pl.ds(c * CHUNK, CHUNK)],)
    ]
    for cp in cps:
        cp.wait()
    pltpu.sync_copy(urows_v, tgt_emb_out.at[pl.ds(base, BPW)])

    def group_body(grp, carry):
        gofs = (base + grp * G) * L
        pltpu.sync_copy(hist_flat.at[pl.ds(gofs, G * L)], idx_v)
        cps = [
            pltpu.async_copy(
                item_tab.at[idx_v.at[pl.ds(c * CHUNK, CHUNK)]],
                rows_v.at[pl.ds(c * CHUNK, CHUNK)], sem)
            for c in range(NCH)
        ]
        for cp in cps:
            cp.wait()
        for g in range(G):
            rbase = g * L

            def red(l, accs):
                a0, a1, a2, a3 = accs
                o = rbase + l * 8
                a0 = a0 + rows_v[o, :] + rows_v[o + 4, :]
                a1 = a1 + rows_v[o + 1, :] + rows_v[o + 5, :]
                a2 = a2 + rows_v[o + 2, :] + rows_v[o + 6, :]
                a3 = a3 + rows_v[o + 3, :] + rows_v[o + 7, :]
                return (a0, a1, a2, a3)

            z = jnp.zeros((D,), jnp.float32)
            a0, a1, a2, a3 = lax.fori_loop(0, L // 8, red, (z, z, z, z))
            acc_v[g, :] = (a0 + a1) + (a2 + a3)
        pltpu.sync_copy(acc_v, hist_sum_out.at[pl.ds(base + grp * G, G)])
        return carry

    lax.fori_loop(0, NG, group_body, 0)


BLK = 2048


def _tc_mlp_body(hist_sum_ref, user_emb_ref, tgt_emb_ref, hlen_ref,
                 w1_ref, b1_ref, w2_ref, b2_ref,
                 user_out_ref, item_out_ref):
    denom = jnp.maximum(hlen_ref[...], 1).astype(jnp.float32)
    hist_mean = hist_sum_ref[...] / denom
    ui = jnp.concatenate([user_emb_ref[...], hist_mean], axis=-1)
    h = jnp.dot(ui, w1_ref[...], preferred_element_type=jnp.float32)
    h = jnp.maximum(h + b1_ref[...], 0.0)
    uv = jnp.dot(h, w2_ref[...], preferred_element_type=jnp.float32)
    uv = jnp.maximum(uv + b2_ref[...], 0.0)
    un = jnp.sqrt(jnp.sum(uv * uv, axis=-1, keepdims=True))
    user_out_ref[...] = uv / jnp.maximum(un, 1e-12)
    tv = tgt_emb_ref[...]
    tn = jnp.sqrt(jnp.sum(tv * tv, axis=-1, keepdims=True))
    item_out_ref[...] = tv / jnp.maximum(tn, 1e-12)


_tc_mlp = pl.pallas_call(
    _tc_mlp_body,
    grid=(B // BLK,),
    in_specs=[
        pl.BlockSpec((BLK, D), lambda i: (i, 0)),
        pl.BlockSpec((BLK, D), lambda i: (i, 0)),
        pl.BlockSpec((BLK, D), lambda i: (i, 0)),
        pl.BlockSpec((BLK, 1), lambda i: (i, 0)),
        pl.BlockSpec((2 * D, 64), lambda i: (0, 0)),
        pl.BlockSpec((1, 64), lambda i: (0, 0)),
        pl.BlockSpec((64, D), lambda i: (0, 0)),
        pl.BlockSpec((1, D), lambda i: (0, 0)),
    ],
    out_specs=[
        pl.BlockSpec((BLK, D), lambda i: (i, 0)),
        pl.BlockSpec((BLK, D), lambda i: (i, 0)),
    ],
    out_shape=[
        jax.ShapeDtypeStruct((B, D), jnp.float32),
        jax.ShapeDtypeStruct((B, D), jnp.float32),
    ],
)


def kernel(user_id, hist_item, hist_len, target_item, user_table, item_table,
           W1, b1, W2, b2):
    hist_flat = hist_item.reshape(-1)
    hist_sum, user_emb, tgt_emb = _sc_gather_pool(
        hist_flat, user_id, target_item, item_table, user_table)
    user_vec, item_vec = _tc_mlp(
        hist_sum, user_emb, tgt_emb, hist_len.reshape(-1, 1),
        W1, b1.reshape(1, -1), W2, b2.reshape(1, -1))
    return (user_vec, item_vec)


# speedup vs baseline: 5.9286x; 1.0937x over previous
import functools

import jax
import jax.numpy as jnp
from jax import lax
from jax.experimental import pallas as pl
from jax.experimental.pallas import tpu as pltpu
from jax.experimental.pallas import tpu_sc as plsc

B = 16384
L = 200
D = 16
NC = 2
NS = 16
NW = NC * NS
BPW = B // NW
G = 16
NG = BPW // G
CHUNK = 128
NCH = (G * L) // CHUNK

_sc_mesh = plsc.VectorSubcoreMesh(core_axis_name="c", subcore_axis_name="s")


@functools.partial(
    pl.kernel,
    mesh=_sc_mesh,
    compiler_params=pltpu.CompilerParams(use_tc_tiling_on_sc=False),
    out_type=(
        jax.ShapeDtypeStruct((B, D), jnp.float32),
        jax.ShapeDtypeStruct((B, D), jnp.float32),
        jax.ShapeDtypeStruct((B, D), jnp.float32),
    ),
    scratch_types=[
        pltpu.VMEM((G * L,), jnp.int32),
        pltpu.VMEM((G * L,), jnp.int32),
        pltpu.VMEM((G * L, D), jnp.float32),
        pltpu.VMEM((G * L, D), jnp.float32),
        pltpu.VMEM((G, D), jnp.float32),
        pltpu.VMEM((BPW,), jnp.int32),
        pltpu.VMEM((BPW, D), jnp.float32),
        pltpu.SemaphoreType.DMA,
        pltpu.SemaphoreType.DMA,
    ],
)
def _sc_gather_pool(hist_flat, user_id, tgt_id, item_tab, user_tab,
                    hist_sum_out, user_emb_out, tgt_emb_out,
                    idx_a, idx_b, rows_a, rows_b, acc_v, uidx_v, urows_v,
                    sem_a, sem_b):
    wid = lax.axis_index("s") * NC + lax.axis_index("c")
    base = wid * BPW

    def small_gather(ids_hbm, tab_hbm, out_hbm):
        pltpu.sync_copy(ids_hbm.at[pl.ds(base, BPW)], uidx_v)
        cps = [
            pltpu.async_copy(
                tab_hbm.at[uidx_v.at[pl.ds(c * CHUNK, CHUNK)]],
                urows_v.at[pl.ds(c * CHUNK, CHUNK)], sem_a)
            for c in range(BPW // CHUNK)
        ]
        for cp in cps:
            cp.wait()
        pltpu.sync_copy(urows_v, out_hbm.at[pl.ds(base, BPW)])

    small_gather(user_id, user_tab, user_emb_out)
    small_gather(tgt_id, item_tab, tgt_emb_out)

    def load_idx(grp, idx_v):
        pltpu.sync_copy(hist_flat.at[pl.ds((base + grp * G) * L, G * L)],
                        idx_v)

    def fire(idx_v, rows_v, sem):
        for c in range(NCH):
            pltpu.async_copy(
                item_tab.at[idx_v.at[pl.ds(c * CHUNK, CHUNK)]],
                rows_v.at[pl.ds(c * CHUNK, CHUNK)], sem)

    def drain(rows_v, sem):
        pltpu.make_async_copy(item_tab.at[pl.ds(0, G * L)], rows_v,
                              sem).wait()

    def reduce_store(rows_v, grp):
        for g in range(G):
            rbase = g * L

            def red(l, accs):
                a0, a1, a2, a3 = accs
                o = rbase + l * 8
                a0 = a0 + rows_v[o, :] + rows_v[o + 4, :]
                a1 = a1 + rows_v[o + 1, :] + rows_v[o + 5, :]
                a2 = a2 + rows_v[o + 2, :] + rows_v[o + 6, :]
                a3 = a3 + rows_v[o + 3, :] + rows_v[o + 7, :]
                return (a0, a1, a2, a3)

            z = jnp.zeros((D,), jnp.float32)
            a0, a1, a2, a3 = lax.fori_loop(0, L // 8, red, (z, z, z, z))
            acc_v[g, :] = (a0 + a1) + (a2 + a3)
        pltpu.sync_copy(acc_v, hist_sum_out.at[pl.ds(base + grp * G, G)])

    load_idx(0, idx_a)
    fire(idx_a, rows_a, sem_a)

    def pair_body(p, carry):
        load_idx(2 * p + 1, idx_b)
        fire(idx_b, rows_b, sem_b)
        drain(rows_a, sem_a)
        reduce_store(rows_a, 2 * p)

        @pl.when(p < NG // 2 - 1)
        def _():
            load_idx(2 * p + 2, idx_a)
            fire(idx_a, rows_a, sem_a)

        drain(rows_b, sem_b)
        reduce_store(rows_b, 2 * p + 1)
        return carry

    lax.fori_loop(0, NG // 2, pair_body, 0)


BLK = 2048


def _tc_mlp_body(hist_sum_ref, user_emb_ref, tgt_emb_ref, hlen_ref,
                 w1_ref, b1_ref, w2_ref, b2_ref,
                 user_out_ref, item_out_ref):
    denom = jnp.maximum(hlen_ref[...], 1).astype(jnp.float32)
    hist_mean = hist_sum_ref[...] / denom
    ui = jnp.concatenate([user_emb_ref[...], hist_mean], axis=-1)
    h = jnp.dot(ui, w1_ref[...], preferred_element_type=jnp.float32)
    h = jnp.maximum(h + b1_ref[...], 0.0)
    uv = jnp.dot(h, w2_ref[...], preferred_element_type=jnp.float32)
    uv = jnp.maximum(uv + b2_ref[...], 0.0)
    un = jnp.sqrt(jnp.sum(uv * uv, axis=-1, keepdims=True))
    user_out_ref[...] = uv / jnp.maximum(un, 1e-12)
    tv = tgt_emb_ref[...]
    tn = jnp.sqrt(jnp.sum(tv * tv, axis=-1, keepdims=True))
    item_out_ref[...] = tv / jnp.maximum(tn, 1e-12)


_tc_mlp = pl.pallas_call(
    _tc_mlp_body,
    grid=(B // BLK,),
    in_specs=[
        pl.BlockSpec((BLK, D), lambda i: (i, 0)),
        pl.BlockSpec((BLK, D), lambda i: (i, 0)),
        pl.BlockSpec((BLK, D), lambda i: (i, 0)),
        pl.BlockSpec((BLK, 1), lambda i: (i, 0)),
        pl.BlockSpec((2 * D, 64), lambda i: (0, 0)),
        pl.BlockSpec((1, 64), lambda i: (0, 0)),
        pl.BlockSpec((64, D), lambda i: (0, 0)),
        pl.BlockSpec((1, D), lambda i: (0, 0)),
    ],
    out_specs=[
        pl.BlockSpec((BLK, D), lambda i: (i, 0)),
        pl.BlockSpec((BLK, D), lambda i: (i, 0)),
    ],
    out_shape=[
        jax.ShapeDtypeStruct((B, D), jnp.float32),
        jax.ShapeDtypeStruct((B, D), jnp.float32),
    ],
)


def kernel(user_id, hist_item, hist_len, target_item, user_table, item_table,
           W1, b1, W2, b2):
    hist_flat = hist_item.reshape(-1)
    hist_sum, user_emb, tgt_emb = _sc_gather_pool(
        hist_flat, user_id, target_item, item_table, user_table)
    user_vec, item_vec = _tc_mlp(
        hist_sum, user_emb, tgt_emb, hist_len.reshape(-1, 1),
        W1, b1.reshape(1, -1), W2, b2.reshape(1, -1))
    return (user_vec, item_vec)


# speedup vs baseline: 6.5902x; 1.1116x over previous
import functools

import jax
import jax.numpy as jnp
from jax import lax
from jax.experimental import pallas as pl
from jax.experimental.pallas import tpu as pltpu
from jax.experimental.pallas import tpu_sc as plsc

B = 16384
L = 200
D = 16
NC = 2
NS = 16
NW = NC * NS
BPW = B // NW
G = 16
NG = BPW // G
CHUNK = 128
NCH = (G * L) // CHUNK

_sc_mesh = plsc.VectorSubcoreMesh(core_axis_name="c", subcore_axis_name="s")


@functools.partial(
    pl.kernel,
    mesh=_sc_mesh,
    compiler_params=pltpu.CompilerParams(use_tc_tiling_on_sc=False),
    out_type=(
        jax.ShapeDtypeStruct((B, D), jnp.float32),
        jax.ShapeDtypeStruct((B, D), jnp.float32),
    ),
    scratch_types=[
        pltpu.VMEM((G * L,), jnp.int32),
        pltpu.VMEM((G * L,), jnp.int32),
        pltpu.VMEM((G * L, D), jnp.float32),
        pltpu.VMEM((G * L, D), jnp.float32),
        pltpu.VMEM((G, D), jnp.float32),
        pltpu.VMEM((BPW,), jnp.int32),
        pltpu.VMEM((BPW, D), jnp.float32),
        pltpu.SemaphoreType.DMA,
        pltpu.SemaphoreType.DMA,
    ],
)
def _sc_gather_pool(hist_flat, tgt_id, item_tab,
                    hist_sum_out, tgt_emb_out,
                    idx_a, idx_b, rows_a, rows_b, acc_v, uidx_v, urows_v,
                    sem_a, sem_b):
    wid = lax.axis_index("s") * NC + lax.axis_index("c")
    base = wid * BPW

    def small_gather(ids_hbm, tab_hbm, out_hbm):
        pltpu.sync_copy(ids_hbm.at[pl.ds(base, BPW)], uidx_v)
        cps = [
            pltpu.async_copy(
                tab_hbm.at[uidx_v.at[pl.ds(c * CHUNK, CHUNK)]],
                urows_v.at[pl.ds(c * CHUNK, CHUNK)], sem_a)
            for c in range(BPW // CHUNK)
        ]
        for cp in cps:
            cp.wait()
        pltpu.sync_copy(urows_v, out_hbm.at[pl.ds(base, BPW)])

    small_gather(tgt_id, item_tab, tgt_emb_out)

    def load_idx(grp, idx_v):
        pltpu.sync_copy(hist_flat.at[pl.ds((base + grp * G) * L, G * L)],
                        idx_v)

    def fire(idx_v, rows_v, sem):
        for c in range(NCH):
            pltpu.async_copy(
                item_tab.at[idx_v.at[pl.ds(c * CHUNK, CHUNK)]],
                rows_v.at[pl.ds(c * CHUNK, CHUNK)], sem)

    def drain(rows_v, sem):
        pltpu.make_async_copy(item_tab.at[pl.ds(0, G * L)], rows_v,
                              sem).wait()

    def reduce_store(rows_v, grp):
        for g in range(G):
            rbase = g * L

            def red(l, accs):
                a0, a1, a2, a3 = accs
                o = rbase + l * 8
                a0 = a0 + rows_v[o, :] + rows_v[o + 4, :]
                a1 = a1 + rows_v[o + 1, :] + rows_v[o + 5, :]
                a2 = a2 + rows_v[o + 2, :] + rows_v[o + 6, :]
                a3 = a3 + rows_v[o + 3, :] + rows_v[o + 7, :]
                return (a0, a1, a2, a3)

            z = jnp.zeros((D,), jnp.float32)
            a0, a1, a2, a3 = lax.fori_loop(0, L // 8, red, (z, z, z, z))
            acc_v[g, :] = (a0 + a1) + (a2 + a3)
        pltpu.sync_copy(acc_v, hist_sum_out.at[pl.ds(base + grp * G, G)])

    load_idx(0, idx_a)
    fire(idx_a, rows_a, sem_a)

    def pair_body(p, carry):
        load_idx(2 * p + 1, idx_b)
        fire(idx_b, rows_b, sem_b)
        drain(rows_a, sem_a)
        reduce_store(rows_a, 2 * p)

        @pl.when(p < NG // 2 - 1)
        def _():
            load_idx(2 * p + 2, idx_a)
            fire(idx_a, rows_a, sem_a)

        drain(rows_b, sem_b)
        reduce_store(rows_b, 2 * p + 1)
        return carry

    lax.fori_loop(0, NG // 2, pair_body, 0)


UT_TILES = 1000000 // 8


@functools.partial(
    pl.kernel,
    mesh=_sc_mesh,
    compiler_params=pltpu.CompilerParams(use_tc_tiling_on_sc=True,
                                         needs_layout_passes=False),
    out_type=jax.ShapeDtypeStruct((B, D), jnp.float32),
    scratch_types=[
        pltpu.VMEM((BPW,), jnp.int32),
        pltpu.VMEM((BPW,), jnp.int32),
        pltpu.VMEM((CHUNK, 128), jnp.float32),
        pltpu.VMEM((BPW, D), jnp.float32),
        pltpu.SemaphoreType.DMA,
    ],
)
def _sc_user_gather(user_id, user_tab8, user_emb_out,
                    ids_v, tid_v, tiles_v, out_v, sem):
    wid = lax.axis_index("s") * NC + lax.axis_index("c")
    base = wid * BPW
    pltpu.sync_copy(user_id.at[pl.ds(base, BPW)], ids_v)
    for k in range(BPW // 16):
        iv = ids_v[pl.ds(k * 16, 16)]
        tid_v[pl.ds(k * 16, 16)] = iv >> 3
    lane = lax.iota(jnp.int32, 16)
    for c in range(BPW // CHUNK):
        pltpu.async_copy(
            user_tab8.at[tid_v.at[pl.ds(c * CHUNK, CHUNK)]],
            tiles_v, sem).wait()
        for k in range(CHUNK // 16):
            iv = ids_v[pl.ds(c * CHUNK + k * 16, 16)]
            j_vec = lane + (k * 16)
            row_vec = j_vec + c * CHUNK
            off = (iv & 7) * D
            for jj in range(D):
                vals = plsc.load_gather(tiles_v, [j_vec, off + jj])
                plsc.store_scatter(out_v, [row_vec, lane * 0 + jj], vals)
    pltpu.sync_copy(out_v, user_emb_out.at[pl.ds(base, BPW)])


BLK = 2048


def _tc_mlp_body(hist_sum_ref, user_emb_ref, tgt_emb_ref, hlen_ref,
                 w1_ref, b1_ref, w2_ref, b2_ref,
                 user_out_ref, item_out_ref):
    denom = jnp.maximum(hlen_ref[...], 1).astype(jnp.float32)
    hist_mean = hist_sum_ref[...] / denom
    ui = jnp.concatenate([user_emb_ref[...], hist_mean], axis=-1)
    h = jnp.dot(ui, w1_ref[...], preferred_element_type=jnp.float32)
    h = jnp.maximum(h + b1_ref[...], 0.0)
    uv = jnp.dot(h, w2_ref[...], preferred_element_type=jnp.float32)
    uv = jnp.maximum(uv + b2_ref[...], 0.0)
    un = jnp.sqrt(jnp.sum(uv * uv, axis=-1, keepdims=True))
    user_out_ref[...] = uv / jnp.maximum(un, 1e-12)
    tv = tgt_emb_ref[...]
    tn = jnp.sqrt(jnp.sum(tv * tv, axis=-1, keepdims=True))
    item_out_ref[...] = tv / jnp.maximum(tn, 1e-12)


_tc_mlp = pl.pallas_call(
    _tc_mlp_body,
    grid=(B // BLK,),
    in_specs=[
        pl.BlockSpec((BLK, D), lambda i: (i, 0)),
        pl.BlockSpec((BLK, D), lambda i: (i, 0)),
        pl.BlockSpec((BLK, D), lambda i: (i, 0)),
        pl.BlockSpec((BLK, 1), lambda i: (i, 0)),
        pl.BlockSpec((2 * D, 64), lambda i: (0, 0)),
        pl.BlockSpec((1, 64), lambda i: (0, 0)),
        pl.BlockSpec((64, D), lambda i: (0, 0)),
        pl.BlockSpec((1, D), lambda i: (0, 0)),
    ],
    out_specs=[
        pl.BlockSpec((BLK, D), lambda i: (i, 0)),
        pl.BlockSpec((BLK, D), lambda i: (i, 0)),
    ],
    out_shape=[
        jax.ShapeDtypeStruct((B, D), jnp.float32),
        jax.ShapeDtypeStruct((B, D), jnp.float32),
    ],
)


def kernel(user_id, hist_item, hist_len, target_item, user_table, item_table,
           W1, b1, W2, b2):
    hist_flat = hist_item.reshape(-1)
    hist_sum, tgt_emb = _sc_gather_pool(hist_flat, target_item, item_table)
    user_emb = _sc_user_gather(user_id, user_table.reshape(UT_TILES, 128))
    user_vec, item_vec = _tc_mlp(
        hist_sum, user_emb, tgt_emb, hist_len.reshape(-1, 1),
        W1, b1.reshape(1, -1), W2, b2.reshape(1, -1))
    return (user_vec, item_vec)
